# baseline (speedup 1.0000x reference)
import functools

import jax
import jax.numpy as jnp
from jax import lax
from jax.experimental import layout as jlayout
from jax.experimental import pallas as pl
from jax.experimental.pallas import tpu as pltpu
from jax.experimental.pallas import tpu_sc as plsc

ROW = 128
GROUP_ROWS = 4
GS = ROW * GROUP_ROWS
NBUF = 2


@functools.partial(jax.jit, static_argnums=(2, 3, 4))
def _embed(x2, table, N, NC, NS):
    D = table.shape[1]
    NW = NC * NS
    per_w = N // NW
    n_groups = per_w // GS
    assert n_groups % 2 == 0 and n_groups >= 4
    mesh = plsc.VectorSubcoreMesh(core_axis_name="c", subcore_axis_name="s")

    @functools.partial(
        pl.kernel,
        mesh=mesh,
        out_type=jax.ShapeDtypeStruct((N, D), jnp.float32),
        compiler_params=pltpu.CompilerParams(use_tc_tiling_on_sc=False),
        scratch_types=[
            pltpu.VMEM((NBUF, GROUP_ROWS, ROW), jnp.int32),
            pltpu.VMEM((NBUF, GS, D), jnp.float32),
            pltpu.SemaphoreType.DMA,
            pltpu.SemaphoreType.DMA,
        ],
    )
    def k(x_hbm, table_hbm, out_hbm, idx_v, rows_v, sem0, sem1):
        wid = lax.axis_index("s") * NC + lax.axis_index("c")
        wb = wid * per_w
        sems = [sem0, sem1]

        def issue(g, b):
            base = pl.multiple_of(wb + g * GS, GS)
            row0 = pl.multiple_of(base // ROW, GROUP_ROWS)
            pltpu.sync_copy(x_hbm.at[pl.ds(row0, GROUP_ROWS)], idx_v.at[b])
            for j in range(GROUP_ROWS):
                pltpu.async_copy(
                    table_hbm.at[idx_v.at[b, j]],
                    rows_v.at[b, pl.ds(j * ROW, ROW)],
                    sems[b],
                )

        def drain(g, b):
            pltpu.make_async_copy(
                out_hbm.at[pl.ds(0, GS)], rows_v.at[b], sems[b]
            ).wait()
            base = pl.multiple_of(wb + g * GS, GS)
            pltpu.sync_copy(rows_v.at[b], out_hbm.at[pl.ds(base, GS)])

        issue(0, 0)

        def body(i, carry):
            g = 2 * i
            issue(g + 1, 1)
            drain(g, 0)
            issue(g + 2, 0)
            drain(g + 1, 1)
            return carry

        lax.fori_loop(0, n_groups // 2 - 1, body, 0)
        g_last = n_groups - 1
        issue(g_last, 1)
        drain(g_last - 1, 0)
        drain(g_last, 1)

    return k(x2, table)


def kernel(x, table):
    B, H = x.shape
    D = table.shape[1]
    N = B * H
    info = plsc.get_sparse_core_info()
    NC, NS = info.num_cores, info.num_subcores
    assert N % (NC * NS * GS) == 0
    x2 = x.astype(jnp.int32).reshape(N // ROW, ROW)
    out = _embed(x2, table, N, NC, NS)
    return jlayout.with_layout_constraint(
        out.reshape(B, H, D), jlayout.Layout((1, 2, 0), ((8, 128),)))
